# baseline (speedup 1.0000x reference)
import jax
import jax.numpy as jnp
from jax import lax
from jax.experimental import pallas as pl
from jax.experimental.pallas import tpu as pltpu
from jax.experimental.pallas import tpu_sc as plsc

B, L, C = 16384, 50, 1000
NC, NS = 2, 16
NW = NC * NS
NQ = 4
NT = NW // NQ
CC = 256
RB = 128
NBLK = B // (NT * RB)
NRING = 4


def _body(names_hbm, out_hbm, nm0, nm1, nm2, nm3, bufa, bufb,
          sn0, sn1, sn2, sn3, soa, sob):
    wid = lax.axis_index("s") * NC + lax.axis_index("c")
    team = wid // NQ
    q = wid % NQ
    c0 = jnp.where(q == NQ - 1, C - CC, q * CC)

    nslots = (nm0, nm1, nm2, nm3)
    nsems = (sn0, sn1, sn2, sn3)

    def r0_of(j):
        return (team * NBLK + j) * RB

    def names_cp(j):
        return pltpu.make_async_copy(
            names_hbm.at[:, pl.ds(r0_of(j), RB)],
            nslots[j % NRING], nsems[j % NRING])

    def out_cp(buf, j, sem):
        return pltpu.make_async_copy(
            buf, out_hbm.at[pl.ds(c0, CC), pl.ds(r0_of(j), RB)], sem)

    for j in range(NRING):
        names_cp(j).start()

    zeros = jnp.zeros((16,), jnp.float32)
    ones = jnp.full((16,), 1.0, jnp.float32)
    iota = lax.iota(jnp.int32, 16)
    cc_u = jnp.uint32(CC)

    def zero_body(ci, _):
        for k in range(RB // 16):
            bufa[ci, pl.ds(k * 16, 16)] = zeros
            bufb[ci, pl.ds(k * 16, 16)] = zeros
        return 0
    lax.fori_loop(0, CC, zero_body, 0)

    def scan_pass(buf, nm, val_vec):
        @plsc.parallel_loop(0, L, unroll=2)
        def _(l):
            for rs in range(RB // 16):
                nv = nm[l, pl.ds(rs * 16, 16)]
                cv = nv - c0
                mask = cv.astype(jnp.uint32) < cc_u
                plsc.store_scatter(buf, [cv, rs * 16 + iota], val_vec,
                                   mask=mask)

    for j in range(NBLK):
        buf, sem = (bufa, soa) if j % 2 == 0 else (bufb, sob)
        if j >= 2:
            out_cp(buf, j - 2, sem).wait()
            scan_pass(buf, nslots[(j - 2) % NRING], zeros)
            if j + 2 < NBLK:
                names_cp(j + 2).start()
        names_cp(j).wait()
        scan_pass(buf, nslots[j % NRING], ones)
        out_cp(buf, j, sem).start()

    out_cp(bufa, NBLK - 2, soa).wait()
    out_cp(bufb, NBLK - 1, sob).wait()


@jax.jit
def kernel(names):
    mesh = plsc.VectorSubcoreMesh(
        core_axis_name="c", subcore_axis_name="s",
        num_cores=NC, num_subcores=NS)
    out_t = pl.kernel(
        _body,
        out_type=jax.ShapeDtypeStruct((C, B), jnp.float32),
        mesh=mesh,
        compiler_params=pltpu.CompilerParams(
            needs_layout_passes=False, skip_device_barrier=True),
        scratch_types=[
            pltpu.VMEM((L, RB), jnp.int32),
            pltpu.VMEM((L, RB), jnp.int32),
            pltpu.VMEM((L, RB), jnp.int32),
            pltpu.VMEM((L, RB), jnp.int32),
            pltpu.VMEM((CC, RB), jnp.float32),
            pltpu.VMEM((CC, RB), jnp.float32),
            pltpu.SemaphoreType.DMA,
            pltpu.SemaphoreType.DMA,
            pltpu.SemaphoreType.DMA,
            pltpu.SemaphoreType.DMA,
            pltpu.SemaphoreType.DMA,
            pltpu.SemaphoreType.DMA,
        ],
    )(names.T)
    return out_t.T


# speedup vs baseline: 1.0063x; 1.0063x over previous
import jax
import jax.numpy as jnp
from jax import lax
from jax.experimental import pallas as pl
from jax.experimental.pallas import tpu as pltpu
from jax.experimental.pallas import tpu_sc as plsc

B, L, C = 16384, 50, 1000
NC, NS = 2, 16
NW = NC * NS
NQ = 4
NT = NW // NQ
CC = 256
RB = 128
NBLK = B // (NT * RB)
NRING = 4


def _body(names_hbm, out_hbm, nm0, nm1, nm2, nm3, bufa, bufb,
          sn0, sn1, sn2, sn3, soa, sob):
    wid = lax.axis_index("s") * NC + lax.axis_index("c")
    team = wid // NQ
    q = wid % NQ
    c0 = jnp.where(q == NQ - 1, C - CC, q * CC)

    nslots = (nm0, nm1, nm2, nm3)
    nsems = (sn0, sn1, sn2, sn3)

    def r0_of(j):
        return (team * NBLK + j) * RB

    def names_cp(j):
        return pltpu.make_async_copy(
            names_hbm.at[:, pl.ds(r0_of(j), RB)],
            nslots[j % NRING], nsems[j % NRING])

    def out_cp(buf, j, sem):
        return pltpu.make_async_copy(
            buf, out_hbm.at[pl.ds(c0, CC), pl.ds(r0_of(j), RB)], sem)

    for j in range(NRING):
        names_cp(j).start()

    zeros = jnp.zeros((16,), jnp.float32)
    ones = jnp.full((16,), 1.0, jnp.float32)
    iota = lax.iota(jnp.int32, 16)
    cc_u = jnp.uint32(CC)

    def zero_buf(buf):
        def zb(ci, _):
            for k in range(RB // 16):
                buf[ci, pl.ds(k * 16, 16)] = zeros
            return 0
        lax.fori_loop(0, CC, zb, 0)

    def scan_pass(buf, nm, val_vec):
        @plsc.parallel_loop(0, L, unroll=2)
        def _(l):
            for rs in range(RB // 16):
                nv = nm[l, pl.ds(rs * 16, 16)]
                cv = nv - c0
                mask = cv.astype(jnp.uint32) < cc_u
                plsc.store_scatter(buf, [cv, rs * 16 + iota], val_vec,
                                   mask=mask)

    zero_buf(bufa)
    names_cp(0).wait()
    scan_pass(bufa, nslots[0], ones)
    out_cp(bufa, 0, soa).start()
    zero_buf(bufb)
    names_cp(1).wait()
    scan_pass(bufb, nslots[1], ones)
    out_cp(bufb, 1, sob).start()

    for j in range(2, NBLK):
        buf, sem = (bufa, soa) if j % 2 == 0 else (bufb, sob)
        out_cp(buf, j - 2, sem).wait()
        scan_pass(buf, nslots[(j - 2) % NRING], zeros)
        if j + 2 < NBLK:
            names_cp(j + 2).start()
        names_cp(j).wait()
        scan_pass(buf, nslots[j % NRING], ones)
        out_cp(buf, j, sem).start()

    out_cp(bufa, NBLK - 2, soa).wait()
    out_cp(bufb, NBLK - 1, sob).wait()


@jax.jit
def kernel(names):
    mesh = plsc.VectorSubcoreMesh(
        core_axis_name="c", subcore_axis_name="s",
        num_cores=NC, num_subcores=NS)
    out_t = pl.kernel(
        _body,
        out_type=jax.ShapeDtypeStruct((C, B), jnp.float32),
        mesh=mesh,
        compiler_params=pltpu.CompilerParams(needs_layout_passes=False),
        scratch_types=[
            pltpu.VMEM((L, RB), jnp.int32),
            pltpu.VMEM((L, RB), jnp.int32),
            pltpu.VMEM((L, RB), jnp.int32),
            pltpu.VMEM((L, RB), jnp.int32),
            pltpu.VMEM((CC, RB), jnp.float32),
            pltpu.VMEM((CC, RB), jnp.float32),
            pltpu.SemaphoreType.DMA,
            pltpu.SemaphoreType.DMA,
            pltpu.SemaphoreType.DMA,
            pltpu.SemaphoreType.DMA,
            pltpu.SemaphoreType.DMA,
            pltpu.SemaphoreType.DMA,
        ],
    )(names.T)
    return out_t.T
